# Initial kernel scaffold; baseline (speedup 1.0000x reference)
import jax
import jax.numpy as jnp
from jax.experimental import pallas as pl


def kernel(target, context, target_table, context_table):
    raise NotImplementedError("write your pallas kernel here")


# speedup vs baseline: 4.2389x; 4.2389x over previous
import dataclasses
import functools

import jax
import jax.numpy as jnp
from jax import lax
from jax.experimental import pallas as pl
from jax.experimental.pallas import tpu as pltpu
from jax.experimental.pallas import tpu_sc as plsc

B = 4096
C = 6
E = 128
L = 16
NC = 2
NS = 16
NW = NC * NS
BPW = B // NW
NCH = 4
RPC = BPW // NCH
CPC = RPC * C
HALF = CPC // 2


def _dots_sc(target, ctx_flat, target_table, context_table):
  mesh = plsc.VectorSubcoreMesh(core_axis_name="c", subcore_axis_name="s")

  cp = pltpu.CompilerParams()
  if "needs_layout_passes" in pltpu.CompilerParams.__dataclass_fields__:
    cp = dataclasses.replace(cp, needs_layout_passes=False)

  @functools.partial(
      pl.kernel,
      compiler_params=cp,
      out_type=jax.ShapeDtypeStruct((B * C,), jnp.float32),
      mesh=mesh,
      scratch_types=[
          pltpu.VMEM((BPW,), jnp.int32),
          pltpu.VMEM((BPW * C,), jnp.int32),
          pltpu.VMEM((2, RPC, E), jnp.float32),
          pltpu.VMEM((2, CPC, E), jnp.float32),
          pltpu.VMEM((BPW * C,), jnp.float32),
          pltpu.SemaphoreType.DMA,
          pltpu.SemaphoreType.DMA,
      ],
  )
  def k(tgt_hbm, ctx_hbm, ttab_hbm, ctab_hbm, out_hbm,
        tidx_v, cidx_v, trows_v, crows_v, out_v, sem0, sem1):
    wid = lax.axis_index("c") * NS + lax.axis_index("s")
    base = wid * BPW
    sems = (sem0, sem1)

    pltpu.sync_copy(tgt_hbm.at[pl.ds(base, BPW)], tidx_v)
    pltpu.sync_copy(ctx_hbm.at[pl.ds(base * C, BPW * C)], cidx_v)

    def fire(g, p):
      return [
          pltpu.async_copy(
              ttab_hbm.at[tidx_v.at[pl.ds(g * RPC, RPC)]],
              trows_v.at[p], sems[p]),
          pltpu.async_copy(
              ctab_hbm.at[cidx_v.at[pl.ds(g * CPC, HALF)]],
              crows_v.at[p].at[pl.ds(0, HALF)], sems[p]),
          pltpu.async_copy(
              ctab_hbm.at[cidx_v.at[pl.ds(g * CPC + HALF, HALF)]],
              crows_v.at[p].at[pl.ds(HALF, HALF)], sems[p]),
      ]

    lanes = jax.lax.iota(jnp.int32, L)
    lane_mask = lanes < C

    inflight = fire(0, 0)
    for g in range(NCH):
      p = g % 2
      cur = inflight
      if g + 1 < NCH:
        inflight = fire(g + 1, (g + 1) % 2)
      for cpy in cur:
        cpy.wait()

      tbuf = trows_v.at[p]
      cbuf = crows_v.at[p]
      out_base = (base * 0) + g * RPC * C

      @pl.loop(0, RPC)
      def _(i):
        t = [tbuf[i, pl.ds(kk * L, L)] for kk in range(E // L)]
        res = jnp.zeros((L,), jnp.float32)
        for c in range(C):
          row = i * C + c
          acc = t[0] * cbuf[row, pl.ds(0, L)]
          for kk in range(1, E // L):
            acc = acc + t[kk] * cbuf[row, pl.ds(kk * L, L)]
          res = jnp.where(lanes == c, jnp.sum(acc), res)
        plsc.store_scatter(out_v, [out_base + i * C + lanes], res,
                           mask=lane_mask)

    pltpu.sync_copy(out_v, out_hbm.at[pl.ds(base * C, BPW * C)])

  return k(target, ctx_flat, target_table, context_table)


def kernel(target, context, target_table, context_table):
  ctx_flat = context.reshape(-1)
  out = _dots_sc(target, ctx_flat, target_table, context_table)
  return out.reshape(B, C)
